# Initial kernel scaffold; baseline (speedup 1.0000x reference)
import jax
import jax.numpy as jnp
from jax.experimental import pallas as pl


def kernel(sentence_embeddings, sentence_spans):
    raise NotImplementedError("write your pallas kernel here")


# speedup vs baseline: 1.3554x; 1.3554x over previous
import functools

import jax
import jax.numpy as jnp
from jax import lax
from jax.experimental import pallas as pl
from jax.experimental.pallas import tpu as pltpu
from jax.experimental.pallas import tpu_sc as plsc


def _build_seg_sum(N, D, S):
    NC, NS = 2, 16
    NW = NC * NS
    SPW = (-(-S // NW) + 7) // 8 * 8
    SLICE = (SPW + 1 + 16 + 7) // 8 * 8
    PAD = (NW - 1) * SPW + SLICE
    C = 512
    CB = C + 8
    DV = D // 16

    mesh = plsc.VectorSubcoreMesh(
        core_axis_name="c", subcore_axis_name="s",
        num_cores=NC, num_subcores=NS)

    @functools.partial(
        pl.kernel,
        out_type=jax.ShapeDtypeStruct((NW * SPW, D), jnp.float32),
        mesh=mesh,
        scratch_types=[
            pltpu.VMEM((SLICE,), jnp.int32),
            pltpu.VMEM((SLICE,), jnp.int32),
            pltpu.VMEM((CB, D), jnp.float32),
            pltpu.VMEM((SPW, D), jnp.float32),
        ],
    )
    def seg_sum(x_hbm, starts_hbm, adv_hbm, out_hbm,
                starts_v, adv_v, rows_v, out_v):
        wid = lax.axis_index("s") * NC + lax.axis_index("c")
        s_lo = wid * SPW
        n_s = jnp.maximum(0, jnp.minimum(SPW, S - s_lo))
        pltpu.sync_copy(starts_hbm.at[pl.ds(s_lo, SLICE)], starts_v)
        pltpu.sync_copy(adv_hbm.at[pl.ds(s_lo, SLICE)], adv_v)

        def sread(ref, idx):
            return ref[pl.ds(idx, 16)][0]

        row_lo = sread(starts_v, 0)
        row_hi = sread(starts_v, n_s)
        n_chunks = (row_hi - row_lo + C - 1) // C
        zero = jnp.zeros((16,), jnp.float32)
        sptr_init = jnp.minimum(sread(adv_v, 0) - s_lo, SLICE - 17)

        def zero_body(i, _):
            for j in range(DV):
                out_v[i, pl.ds(16 * j, 16)] = zero
            return 0

        lax.fori_loop(0, SPW, zero_body, 0)

        def chunk_body(k, carry):
            sptr0, acc0 = carry[0], carry[1:]
            base = row_lo + k * C
            base_c = jnp.minimum((base // 8) * 8, N - CB)
            pltpu.sync_copy(x_hbm.at[pl.ds(base_c, CB)], rows_v)
            chunk_end = jnp.minimum(base + C, row_hi)

            def row_body(r, st):
                sptr = st[0]
                acc = st[1:]
                acc = tuple(acc[j] + rows_v[r - base_c, pl.ds(16 * j, 16)]
                            for j in range(DV))
                nxt = sread(starts_v, sptr + 1)
                do_flush = jnp.logical_and(nxt == r + 1, sptr < n_s)

                @pl.when(do_flush)
                def _():
                    for j in range(DV):
                        out_v[sptr, pl.ds(16 * j, 16)] = acc[j]

                jump = jnp.minimum(sread(adv_v, sptr + 1) - s_lo, SLICE - 17)
                sptr = jnp.where(do_flush, jump, sptr)
                acc = tuple(jnp.where(do_flush, zero, a) for a in acc)
                return (sptr,) + acc

            return lax.fori_loop(base, chunk_end, row_body, (sptr0,) + acc0)

        carry0 = (sptr_init,) + (zero,) * DV
        lax.fori_loop(0, n_chunks, chunk_body, carry0)

        pltpu.sync_copy(out_v, out_hbm.at[pl.ds(s_lo, SPW)])

    return seg_sum, SPW, SLICE, PAD


def kernel(sentence_embeddings, sentence_spans):
    x = sentence_embeddings
    N, D = x.shape
    S = sentence_spans.shape[0]

    seg_sum, SPW, SLICE, PAD = _build_seg_sum(N, D, S)

    starts = sentence_spans[:, 0].astype(jnp.int32)
    starts_ext = jnp.concatenate(
        [starts, jnp.full((PAD - S,), N, dtype=jnp.int32)])
    adv_ext = (jnp.searchsorted(starts, starts_ext, side="right")
               .astype(jnp.int32) - 1)

    out_pad = seg_sum(x, starts_ext, adv_ext)
    return out_pad[:S]
